# Initial kernel scaffold; baseline (speedup 1.0000x reference)
import jax
import jax.numpy as jnp
from jax.experimental import pallas as pl


def kernel(h_0, h_1, rel_pos, edge_index, edge_attr_0, edge_attr_1, channel_weights):
    raise NotImplementedError("write your pallas kernel here")


# speedup vs baseline: 27.3829x; 27.3829x over previous
import functools

import jax
import jax.numpy as jnp
from jax import lax
from jax.experimental import pallas as pl
from jax.experimental.pallas import tpu as pltpu
from jax.experimental.pallas import tpu_sc as plsc

_N = 50000
_E = 800000
_C = 16
_CHUNK = 128
_NCHUNK = _E // _CHUNK
_HALF = _N // 2
_ACC_ROWS = 25088
_RPT = _ACC_ROWS // 16

_mesh = plsc.VectorSubcoreMesh(core_axis_name="c", subcore_axis_name="s")


@functools.partial(
    pl.kernel,
    out_type=jax.ShapeDtypeStruct((2 * _ACC_ROWS, 64), jnp.float32),
    mesh=_mesh,
    compiler_params=pltpu.CompilerParams(needs_layout_passes=False,
                                         use_tc_tiling_on_sc=False),
    scratch_types=[
        pltpu.VMEM((_CHUNK,), jnp.int32),
        pltpu.VMEM((_CHUNK,), jnp.int32),
        pltpu.VMEM((_CHUNK,), jnp.int32),
        pltpu.VMEM((_CHUNK, 16), jnp.float32),
        pltpu.VMEM((_CHUNK, 48), jnp.float32),
        pltpu.VMEM((_CHUNK, 64), jnp.float32),
        pltpu.VMEM((_CHUNK, 64), jnp.float32),
        pltpu.VMEM_SHARED((_ACC_ROWS, 64), jnp.float32),
        pltpu.SemaphoreType.DMA,
    ],
)
def _edge_kernel(htab, srcm, nbrm, ea0m, ea1m, out_hbm,
                 nbr_v, src_v, idx_v, ea0_v, ea1_v, g_v, o_v, acc, sem):
    c = lax.axis_index("c")
    s = lax.axis_index("s")
    base = c * _HALF

    zeros16 = jnp.zeros((_C,), jnp.float32)

    def _zero_row(e, _):
        o_v[e, pl.ds(0, 16)] = zeros16
        o_v[e, pl.ds(16, 16)] = zeros16
        o_v[e, pl.ds(32, 16)] = zeros16
        o_v[e, pl.ds(48, 16)] = zeros16
        return 0

    lax.fori_loop(0, _CHUNK, _zero_row, 0)
    row0 = s * _RPT
    for k in range(12):
        pltpu.sync_copy(o_v, acc.at[pl.ds(row0 + k * _CHUNK, _CHUNK)])
    pltpu.sync_copy(o_v.at[pl.ds(0, _RPT - 1536)],
                    acc.at[pl.ds(row0 + 1536, _RPT - 1536)])
    plsc.subcore_barrier()

    iota16 = lax.iota(jnp.int32, _C)
    col_x = iota16 * 3
    col_y = col_x + 1
    col_z = col_x + 2

    nchunks = jnp.where(s < _NCHUNK - 16 * (_NCHUNK // 16),
                        _NCHUNK // 16 + 1, _NCHUNK // 16)

    def chunk_body(n, _):
        j = s + n * 16
        pltpu.sync_copy(nbrm.at[j], nbr_v)
        pltpu.sync_copy(srcm.at[j], src_v)
        pltpu.sync_copy(ea0m.at[j], ea0_v)
        pltpu.sync_copy(ea1m.at[j], ea1_v)
        pltpu.async_copy(htab.at[nbr_v], g_v, sem).wait()

        def idx_body(i, _):
            v = src_v[pl.ds(i * _C, _C)]
            t = v - base
            ok = (t >= 0) & (t < _HALF)
            idx_v[pl.ds(i * _C, _C)] = jnp.where(ok, t, _HALF)
            return 0

        lax.fori_loop(0, _CHUNK // _C, idx_body, 0)

        def edge_body(e, _):
            g0 = g_v[e, pl.ds(0, 16)]
            g1x = g_v[e, pl.ds(16, 16)]
            g1y = g_v[e, pl.ds(32, 16)]
            g1z = g_v[e, pl.ds(48, 16)]
            a0 = ea0_v[e, pl.ds(0, 16)]
            e_splat = jnp.full((_C,), e, jnp.int32)
            a1x = plsc.load_gather(ea1_v, [e_splat, col_x])
            a1y = plsc.load_gather(ea1_v, [e_splat, col_y])
            a1z = plsc.load_gather(ea1_v, [e_splat, col_z])
            o_v[e, pl.ds(0, 16)] = (g0 * a0 + g1x * a1x
                                    + g1y * a1y + g1z * a1z)
            o_v[e, pl.ds(16, 16)] = g0 * a1x + g1x * a0
            o_v[e, pl.ds(32, 16)] = g0 * a1y + g1y * a0
            o_v[e, pl.ds(48, 16)] = g0 * a1z + g1z * a0
            return 0

        lax.fori_loop(0, _CHUNK, edge_body, 0)

        pltpu.sync_copy(o_v, acc.at[idx_v], add=True)
        return 0

    lax.fori_loop(0, nchunks, chunk_body, 0)
    plsc.subcore_barrier()

    pltpu.sync_copy(acc.at[pl.ds(row0, _RPT)],
                    out_hbm.at[pl.ds(c * _ACC_ROWS + row0, _RPT)])


def kernel(h_0, h_1, rel_pos, edge_index, edge_attr_0, edge_attr_1,
           channel_weights):
    del rel_pos, channel_weights
    n = h_0.shape[0]
    htab = jnp.concatenate(
        [h_0, h_1[:, :, 0], h_1[:, :, 1], h_1[:, :, 2]], axis=1)
    src = edge_index[0].astype(jnp.int32).reshape(_NCHUNK, _CHUNK)
    nbr = edge_index[1].astype(jnp.int32).reshape(_NCHUNK, _CHUNK)
    ea0 = edge_attr_0.reshape(_NCHUNK, _CHUNK, 16)
    ea1 = edge_attr_1.reshape(_NCHUNK, _CHUNK, 48)
    out = _edge_kernel(htab, src, nbr, ea0, ea1)
    out = out.reshape(2, _ACC_ROWS, 64)
    full = jnp.concatenate([out[0, :_HALF], out[1, :_HALF]], axis=0)
    out0 = full[:, :16]
    out1 = full[:, 16:].reshape(n, 3, 16).transpose(0, 2, 1)
    return (out0, out1)
